# Initial kernel scaffold; baseline (speedup 1.0000x reference)
import jax
import jax.numpy as jnp
from jax.experimental import pallas as pl


def kernel(support_xyz, support_features, W0_0, b0_0, W0_1, b0_1, W1_0, b1_0, W1_1, b1_1, W2_0, b2_0, W2_1, b2_1, W3_0, b3_0, W3_1, b3_1):
    raise NotImplementedError("write your pallas kernel here")


# speedup vs baseline: 3.0198x; 3.0198x over previous
import functools

import jax
import jax.numpy as jnp
from jax.experimental import pallas as pl
from jax.experimental.pallas import tpu as pltpu

_RADII = (0.1, 0.2, 0.4, 0.8)
_NSAMPLE = 32
_STRIDE = 4


def _stage_body(q_ref, xt_ref, s_ref, w0_ref, b0_ref, w1_ref, b1_ref,
                o_ref, a_ref, *, r2, n, nc, mblk):
    q = q_ref[0]
    sup = s_ref[0]
    nchunks = n // nc

    i0 = jax.lax.broadcasted_iota(jnp.int32, (nc, nc), 0)
    i1 = jax.lax.broadcasted_iota(jnp.int32, (nc, nc), 1)
    lt = jnp.where(i0 <= i1, 1.0, 0.0).astype(jnp.float32)

    qx = q[:, 0:1]
    qy = q[:, 1:2]
    qz = q[:, 2:3]

    def chunk_step(c, cnt):
        sl = pl.ds(c * nc, nc)
        xr = xt_ref[0, 0:1, sl]
        yr = xt_ref[0, 1:2, sl]
        zr = xt_ref[0, 2:3, sl]
        d2 = (qx - xr) ** 2 + (qy - yr) ** 2 + (qz - zr) ** 2
        mask = d2 <= r2
        maskf = jnp.where(mask, 1.0, 0.0).astype(jnp.float32)
        pos = cnt + jax.lax.dot(maskf, lt,
                                preferred_element_type=jnp.float32)
        a_chunk = jnp.where(mask & (pos <= 32.0), pos - 1.0, -1.0)
        a_ref[:, sl] = a_chunk
        return cnt + jnp.sum(maskf, axis=1, keepdims=True)

    cnt0 = jnp.zeros((mblk, 1), jnp.float32)
    if nchunks == 1:
        cnt = chunk_step(0, cnt0)
    else:
        cnt = jax.lax.fori_loop(0, nchunks, chunk_step, cnt0)

    amap = a_ref[:, :]
    vc = jnp.minimum(cnt, 32.0)
    w0 = w0_ref[:, :]
    b0 = b0_ref[:, :]
    w1 = w1_ref[:, :]
    b1 = b1_ref[:, :]

    def slot_val(s_f, row):
        dp = row[:, 0:3] - q
        xin = jnp.concatenate([dp, row[:, 3:]], axis=1)
        h = jnp.maximum(jnp.dot(xin, w0) + b0, 0.0)
        h = jnp.maximum(jnp.dot(h, w1) + b1, 0.0)
        return jnp.where(s_f < vc, h, 0.0)

    def gather_row(s_f):
        oh = jnp.where(amap == s_f, 1.0, 0.0).astype(jnp.float32)
        return jax.lax.dot(oh, sup, precision=jax.lax.Precision.HIGHEST,
                           preferred_element_type=jnp.float32)

    row0 = gather_row(jnp.float32(0.0))
    row0 = jnp.where(cnt < 0.5, sup[0:1, :], row0)
    acc0 = slot_val(jnp.float32(0.0), row0)

    def s_step(s, acc):
        s_f = s.astype(jnp.float32)
        return jnp.maximum(acc, slot_val(s_f, gather_row(s_f)))

    acc = jax.lax.fori_loop(1, 32, s_step, acc0)
    o_ref[0] = acc


def _run_stage(q, xyz, feat, w0, b0, w1, b1, radius):
    b, m, _ = q.shape
    n = xyz.shape[1]
    c1 = w0.shape[1]
    c2 = w1.shape[1]
    d = 3 + feat.shape[2]

    sup = jnp.concatenate([xyz, feat], axis=2)
    xt = jnp.transpose(xyz, (0, 2, 1))
    b0r = b0.reshape(1, c1)
    b1r = b1.reshape(1, c2)

    mblk = min(m, 128)
    nc = min(n, 512)

    body = functools.partial(_stage_body, r2=radius * radius,
                             n=n, nc=nc, mblk=mblk)
    return pl.pallas_call(
        body,
        grid=(b, m // mblk),
        in_specs=[
            pl.BlockSpec((1, mblk, 3), lambda i, j: (i, j, 0)),
            pl.BlockSpec((1, 3, n), lambda i, j: (i, 0, 0)),
            pl.BlockSpec((1, n, d), lambda i, j: (i, 0, 0)),
            pl.BlockSpec((d, c1), lambda i, j: (0, 0)),
            pl.BlockSpec((1, c1), lambda i, j: (0, 0)),
            pl.BlockSpec((c1, c2), lambda i, j: (0, 0)),
            pl.BlockSpec((1, c2), lambda i, j: (0, 0)),
        ],
        out_specs=pl.BlockSpec((1, mblk, c2), lambda i, j: (i, j, 0)),
        out_shape=jax.ShapeDtypeStruct((b, m, c2), jnp.float32),
        scratch_shapes=[pltpu.VMEM((mblk, n), jnp.float32)],
    )(q, xt, sup, w0, b0r, w1, b1r)


def kernel(support_xyz, support_features, W0_0, b0_0, W0_1, b0_1, W1_0, b1_0,
           W1_1, b1_1, W2_0, b2_0, W2_1, b2_1, W3_0, b3_0, W3_1, b3_1):
    params = [
        (W0_0, b0_0, W0_1, b0_1),
        (W1_0, b1_0, W1_1, b1_1),
        (W2_0, b2_0, W2_1, b2_1),
        (W3_0, b3_0, W3_1, b3_1),
    ]
    xyz = support_xyz
    feat = jnp.transpose(support_features, (0, 2, 1))
    for i in range(4):
        n = xyz.shape[1]
        m = n // _STRIDE
        skey = jax.random.fold_in(jax.random.key(42), i)
        samp = jax.random.permutation(skey, n)[:m]
        q = jnp.take(xyz, samp, axis=1)
        w0, b0, w1, b1 = params[i]
        feat = _run_stage(q, xyz, feat, w0, b0, w1, b1, _RADII[i])
        xyz = q
    return jnp.transpose(feat, (0, 2, 1))


# speedup vs baseline: 3.0731x; 1.0176x over previous
import functools

import jax
import jax.numpy as jnp
from jax.experimental import pallas as pl
from jax.experimental.pallas import tpu as pltpu

_RADII = (0.1, 0.2, 0.4, 0.8)
_NSAMPLE = 32
_STRIDE = 4


def _stage_body(q_ref, xt_ref, s_ref, w0_ref, b0_ref, w1_ref, b1_ref,
                o_ref, a_ref, *, r2, n, nc, mblk):
    q = q_ref[0]
    sup = s_ref[0]
    nchunks = n // nc

    i0 = jax.lax.broadcasted_iota(jnp.int32, (nc, nc), 0)
    i1 = jax.lax.broadcasted_iota(jnp.int32, (nc, nc), 1)
    lt = jnp.where(i0 <= i1, 1.0, 0.0).astype(jnp.float32)

    qx = q[:, 0:1]
    qy = q[:, 1:2]
    qz = q[:, 2:3]

    def chunk_step(c, cnt):
        sl = pl.ds(c * nc, nc)
        xr = xt_ref[0, 0:1, sl]
        yr = xt_ref[0, 1:2, sl]
        zr = xt_ref[0, 2:3, sl]
        d2 = (qx - xr) ** 2 + (qy - yr) ** 2 + (qz - zr) ** 2
        mask = d2 <= r2
        maskf = jnp.where(mask, 1.0, 0.0).astype(jnp.float32)
        pos = cnt + jax.lax.dot(maskf, lt,
                                preferred_element_type=jnp.float32)
        a_chunk = jnp.where(mask & (pos <= 32.0), pos - 1.0, -1.0)
        a_ref[:, sl] = a_chunk
        return cnt + jnp.sum(maskf, axis=1, keepdims=True)

    cnt0 = jnp.zeros((mblk, 1), jnp.float32)
    if nchunks == 1:
        cnt = chunk_step(0, cnt0)
    else:
        cnt = jax.lax.fori_loop(0, nchunks, chunk_step, cnt0)

    amap = a_ref[:, :]
    vc = jnp.minimum(cnt, 32.0)
    w0 = w0_ref[:, :]
    b0 = b0_ref[:, :]
    w1 = w1_ref[:, :]
    b1 = b1_ref[:, :]

    def slot_val(s_f, row):
        dp = row[:, 0:3] - q
        xin = jnp.concatenate([dp, row[:, 3:]], axis=1)
        h = jnp.maximum(jnp.dot(xin, w0) + b0, 0.0)
        h = jnp.maximum(jnp.dot(h, w1) + b1, 0.0)
        return jnp.where(s_f < vc, h, 0.0)

    def gather_row(s_f):
        oh = jnp.where(amap == s_f, 1.0, 0.0).astype(jnp.float32)
        return jax.lax.dot(oh, sup, precision=jax.lax.Precision.HIGHEST,
                           preferred_element_type=jnp.float32)

    row0 = gather_row(jnp.float32(0.0))
    row0 = jnp.where(cnt < 0.5, sup[0:1, :], row0)
    acc0 = slot_val(jnp.float32(0.0), row0)

    def s_step(s, acc):
        s_f = s.astype(jnp.float32)
        return jnp.maximum(acc, slot_val(s_f, gather_row(s_f)))

    acc = jax.lax.fori_loop(1, 32, s_step, acc0)
    o_ref[0] = acc


def _run_stage(q, xyz, feat, w0, b0, w1, b1, radius):
    b, m, _ = q.shape
    n = xyz.shape[1]
    c1 = w0.shape[1]
    c2 = w1.shape[1]
    d = 3 + feat.shape[2]

    sup = jnp.concatenate([xyz, feat], axis=2)
    xt = jnp.transpose(xyz, (0, 2, 1))
    b0r = b0.reshape(1, c1)
    b1r = b1.reshape(1, c2)

    mblk = min(m, 128)
    nc = min(n, 512)

    body = functools.partial(_stage_body, r2=radius * radius,
                             n=n, nc=nc, mblk=mblk)
    return pl.pallas_call(
        body,
        grid=(b, m // mblk),
        in_specs=[
            pl.BlockSpec((1, mblk, 3), lambda i, j: (i, j, 0)),
            pl.BlockSpec((1, 3, n), lambda i, j: (i, 0, 0)),
            pl.BlockSpec((1, n, d), lambda i, j: (i, 0, 0)),
            pl.BlockSpec((d, c1), lambda i, j: (0, 0)),
            pl.BlockSpec((1, c1), lambda i, j: (0, 0)),
            pl.BlockSpec((c1, c2), lambda i, j: (0, 0)),
            pl.BlockSpec((1, c2), lambda i, j: (0, 0)),
        ],
        out_specs=pl.BlockSpec((1, mblk, c2), lambda i, j: (i, j, 0)),
        out_shape=jax.ShapeDtypeStruct((b, m, c2), jnp.float32),
        scratch_shapes=[pltpu.VMEM((mblk, n), jnp.float32)],
    )(q, xt, sup, w0, b0r, w1, b1r)


def kernel(support_xyz, support_features, W0_0, b0_0, W0_1, b0_1, W1_0, b1_0,
           W1_1, b1_1, W2_0, b2_0, W2_1, b2_1, W3_0, b3_0, W3_1, b3_1):
    params = [
        (W0_0, b0_0, W0_1, b0_1),
        (W1_0, b1_0, W1_1, b1_1),
        (W2_0, b2_0, W2_1, b2_1),
        (W3_0, b3_0, W3_1, b3_1),
    ]
    xyz = support_xyz
    feat = jnp.transpose(support_features, (0, 2, 1))
    for i in range(4):
        n = xyz.shape[1]
        m = n // _STRIDE
        with jax.ensure_compile_time_eval():
            skey = jax.random.fold_in(jax.random.key(42), i)
            samp = jnp.asarray(jax.random.permutation(skey, n)[:m])
        q = jnp.take(xyz, samp, axis=1)
        w0, b0, w1, b1 = params[i]
        feat = _run_stage(q, xyz, feat, w0, b0, w1, b1, _RADII[i])
        xyz = q
    return jnp.transpose(feat, (0, 2, 1))


# speedup vs baseline: 8.3492x; 2.7169x over previous
import functools

import jax
import jax.numpy as jnp
from jax.experimental import pallas as pl
from jax.experimental.pallas import tpu as pltpu

_RADII = (0.1, 0.2, 0.4, 0.8)
_NSAMPLE = 32
_STRIDE = 4


def _stage_body(q_ref, xt_ref, s_ref, w0_ref, b0_ref, w1_ref, b1_ref,
                o_ref, a_ref, *, r2, n, nc, mblk, d):
    q = q_ref[0]
    sup = s_ref[0]
    nchunks = n // nc

    i0 = jax.lax.broadcasted_iota(jnp.int32, (nc, nc), 0)
    i1 = jax.lax.broadcasted_iota(jnp.int32, (nc, nc), 1)
    lt = jnp.where(i0 <= i1, 1.0, 0.0).astype(jnp.float32)

    qx = q[:, 0:1]
    qy = q[:, 1:2]
    qz = q[:, 2:3]

    def chunk_step(c, cnt):
        sl = pl.ds(c * nc, nc)
        xr = xt_ref[0, 0:1, sl]
        yr = xt_ref[0, 1:2, sl]
        zr = xt_ref[0, 2:3, sl]
        d2 = (qx - xr) ** 2 + (qy - yr) ** 2 + (qz - zr) ** 2
        mask = d2 <= r2
        maskf = jnp.where(mask, 1.0, 0.0).astype(jnp.float32)
        pos = cnt + jax.lax.dot(maskf, lt,
                                preferred_element_type=jnp.float32)
        a_chunk = jnp.where(mask & (pos <= 32.0), pos - 1.0, -1.0)
        a_ref[:, sl] = a_chunk
        return cnt + jnp.sum(maskf, axis=1, keepdims=True)

    cnt0 = jnp.zeros((mblk, 1), jnp.float32)
    if nchunks == 1:
        cnt = chunk_step(0, cnt0)
    else:
        cnt = jax.lax.fori_loop(0, nchunks, chunk_step, cnt0)

    amap = a_ref[:, :]
    vc = jnp.minimum(cnt, 32.0)
    w0 = w0_ref[:, :]
    b0 = b0_ref[:, :]
    w1 = w1_ref[:, :]
    b1 = b1_ref[:, :]

    def slot_val(s_f, row):
        dp = row[:, 0:3] - q
        xin = jnp.concatenate([dp, row[:, 3:]], axis=1)
        h = jnp.maximum(jnp.dot(xin, w0) + b0, 0.0)
        h = jnp.maximum(jnp.dot(h, w1) + b1, 0.0)
        return jnp.where(s_f < vc, h, 0.0)

    def gather_row(s_f):
        oh = jnp.where(amap == s_f, 1.0, 0.0).astype(jnp.bfloat16)
        p = jax.lax.dot(oh, sup, preferred_element_type=jnp.float32)
        return (p[:, :d] + p[:, d:2 * d]) + p[:, 2 * d:]

    row0 = gather_row(jnp.float32(0.0))
    srow0 = sup[0:1, :].astype(jnp.float32)
    srow0 = (srow0[:, :d] + srow0[:, d:2 * d]) + srow0[:, 2 * d:]
    row0 = jnp.where(cnt < 0.5, srow0, row0)
    acc0 = slot_val(jnp.float32(0.0), row0)

    def s_step(s, acc):
        s_f = s.astype(jnp.float32)
        return jnp.maximum(acc, slot_val(s_f, gather_row(s_f)))

    acc = jax.lax.fori_loop(1, 32, s_step, acc0)
    o_ref[0] = acc


def _run_stage(q, xyz, feat, w0, b0, w1, b1, radius):
    b, m, _ = q.shape
    n = xyz.shape[1]
    c1 = w0.shape[1]
    c2 = w1.shape[1]
    d = 3 + feat.shape[2]

    supf = jnp.concatenate([xyz, feat], axis=2)
    sup_hi = supf.astype(jnp.bfloat16)
    res1 = supf - sup_hi.astype(jnp.float32)
    sup_mid = res1.astype(jnp.bfloat16)
    sup_lo = (res1 - sup_mid.astype(jnp.float32)).astype(jnp.bfloat16)
    sup = jnp.concatenate([sup_hi, sup_mid, sup_lo], axis=2)
    xt = jnp.transpose(xyz, (0, 2, 1))
    b0r = b0.reshape(1, c1)
    b1r = b1.reshape(1, c2)

    mblk = min(m, 128)
    nc = min(n, 512)

    body = functools.partial(_stage_body, r2=radius * radius,
                             n=n, nc=nc, mblk=mblk, d=d)
    return pl.pallas_call(
        body,
        grid=(b, m // mblk),
        in_specs=[
            pl.BlockSpec((1, mblk, 3), lambda i, j: (i, j, 0)),
            pl.BlockSpec((1, 3, n), lambda i, j: (i, 0, 0)),
            pl.BlockSpec((1, n, 3 * d), lambda i, j: (i, 0, 0)),
            pl.BlockSpec((d, c1), lambda i, j: (0, 0)),
            pl.BlockSpec((1, c1), lambda i, j: (0, 0)),
            pl.BlockSpec((c1, c2), lambda i, j: (0, 0)),
            pl.BlockSpec((1, c2), lambda i, j: (0, 0)),
        ],
        out_specs=pl.BlockSpec((1, mblk, c2), lambda i, j: (i, j, 0)),
        out_shape=jax.ShapeDtypeStruct((b, m, c2), jnp.float32),
        scratch_shapes=[pltpu.VMEM((mblk, n), jnp.float32)],
    )(q, xt, sup, w0, b0r, w1, b1r)


def kernel(support_xyz, support_features, W0_0, b0_0, W0_1, b0_1, W1_0, b1_0,
           W1_1, b1_1, W2_0, b2_0, W2_1, b2_1, W3_0, b3_0, W3_1, b3_1):
    params = [
        (W0_0, b0_0, W0_1, b0_1),
        (W1_0, b1_0, W1_1, b1_1),
        (W2_0, b2_0, W2_1, b2_1),
        (W3_0, b3_0, W3_1, b3_1),
    ]
    xyz = support_xyz
    feat = jnp.transpose(support_features, (0, 2, 1))
    for i in range(4):
        n = xyz.shape[1]
        m = n // _STRIDE
        skey = jax.random.fold_in(jax.random.key(42), i)
        samp = jax.random.permutation(skey, n)[:m]
        q = jnp.take(xyz, samp, axis=1)
        w0, b0, w1, b1 = params[i]
        feat = _run_stage(q, xyz, feat, w0, b0, w1, b1, _RADII[i])
        xyz = q
    return jnp.transpose(feat, (0, 2, 1))


# speedup vs baseline: 10.6818x; 1.2794x over previous
import functools

import jax
import jax.numpy as jnp
from jax.experimental import pallas as pl
from jax.experimental.pallas import tpu as pltpu

_RADII = (0.1, 0.2, 0.4, 0.8)
_NSAMPLE = 32
_STRIDE = 4


def _stage_body(q_ref, xt_ref, s_ref, w0_ref, b0_ref, w1_ref, b1_ref,
                o_ref, a_ref, *, r2, n, nc, mblk, d):
    q = q_ref[0]
    sup = s_ref[0]
    nchunks = n // nc

    i0 = jax.lax.broadcasted_iota(jnp.int32, (nc, nc), 0)
    i1 = jax.lax.broadcasted_iota(jnp.int32, (nc, nc), 1)
    lt = jnp.where(i0 <= i1, 1.0, 0.0).astype(jnp.float32)

    qx = q[:, 0:1]
    qy = q[:, 1:2]
    qz = q[:, 2:3]

    def chunk_step(c, cnt):
        sl = pl.ds(c * nc, nc)
        xr = xt_ref[0, 0:1, sl]
        yr = xt_ref[0, 1:2, sl]
        zr = xt_ref[0, 2:3, sl]
        d2 = (qx - xr) ** 2 + (qy - yr) ** 2 + (qz - zr) ** 2
        mask = d2 <= r2
        maskf = jnp.where(mask, 1.0, 0.0).astype(jnp.float32)
        pos = cnt + jax.lax.dot(maskf, lt,
                                preferred_element_type=jnp.float32)
        a_chunk = jnp.where(mask & (pos <= 32.0), pos - 1.0, -1.0)
        a_ref[:, sl] = a_chunk
        return cnt + jnp.sum(maskf, axis=1, keepdims=True)

    cnt0 = jnp.zeros((mblk, 1), jnp.float32)
    if nchunks == 1:
        cnt = chunk_step(0, cnt0)
    else:
        cnt = jax.lax.fori_loop(0, nchunks, chunk_step, cnt0)

    amap = a_ref[:, :]
    vc = jnp.minimum(cnt, 32.0)
    w0 = w0_ref[:, :]
    b0 = b0_ref[:, :]
    w1 = w1_ref[:, :]
    b1 = b1_ref[:, :]

    def slot_val(s_f, row):
        dp = row[:, 0:3] - q
        xin = jnp.concatenate([dp, row[:, 3:]], axis=1)
        h = jnp.maximum(jnp.dot(xin, w0) + b0, 0.0)
        h = jnp.maximum(jnp.dot(h, w1) + b1, 0.0)
        return jnp.where(s_f < vc, h, 0.0)

    def gather_row(s_f):
        oh = jnp.where(amap == s_f, 1.0, 0.0).astype(jnp.bfloat16)
        p = jax.lax.dot(oh, sup, preferred_element_type=jnp.float32)
        return (p[:, :d] + p[:, d:2 * d]) + p[:, 2 * d:]

    row0 = gather_row(jnp.float32(0.0))
    srow0 = sup[0:1, :].astype(jnp.float32)
    srow0 = (srow0[:, :d] + srow0[:, d:2 * d]) + srow0[:, 2 * d:]
    row0 = jnp.where(cnt < 0.5, srow0, row0)
    acc0 = slot_val(jnp.float32(0.0), row0)

    def s_step(s, acc):
        s_f = s.astype(jnp.float32)
        return jnp.maximum(acc, slot_val(s_f, gather_row(s_f)))

    acc = jax.lax.fori_loop(1, 32, s_step, acc0)
    o_ref[0] = acc


def _run_stage(q, xyz, feat, w0, b0, w1, b1, radius):
    b, m, _ = q.shape
    n = xyz.shape[1]
    c1 = w0.shape[1]
    c2 = w1.shape[1]
    d = 3 + feat.shape[2]

    supf = jnp.concatenate([xyz, feat], axis=2)
    sup_hi = supf.astype(jnp.bfloat16)
    res1 = supf - sup_hi.astype(jnp.float32)
    sup_mid = res1.astype(jnp.bfloat16)
    sup_lo = (res1 - sup_mid.astype(jnp.float32)).astype(jnp.bfloat16)
    sup = jnp.concatenate([sup_hi, sup_mid, sup_lo], axis=2)
    xt = jnp.transpose(xyz, (0, 2, 1))
    b0r = b0.reshape(1, c1)
    b1r = b1.reshape(1, c2)

    mblk = min(m, 256)
    nc = min(n, 512)

    body = functools.partial(_stage_body, r2=radius * radius,
                             n=n, nc=nc, mblk=mblk, d=d)
    return pl.pallas_call(
        body,
        grid=(b, m // mblk),
        in_specs=[
            pl.BlockSpec((1, mblk, 3), lambda i, j: (i, j, 0)),
            pl.BlockSpec((1, 3, n), lambda i, j: (i, 0, 0)),
            pl.BlockSpec((1, n, 3 * d), lambda i, j: (i, 0, 0)),
            pl.BlockSpec((d, c1), lambda i, j: (0, 0)),
            pl.BlockSpec((1, c1), lambda i, j: (0, 0)),
            pl.BlockSpec((c1, c2), lambda i, j: (0, 0)),
            pl.BlockSpec((1, c2), lambda i, j: (0, 0)),
        ],
        out_specs=pl.BlockSpec((1, mblk, c2), lambda i, j: (i, j, 0)),
        out_shape=jax.ShapeDtypeStruct((b, m, c2), jnp.float32),
        scratch_shapes=[pltpu.VMEM((mblk, n), jnp.float32)],
    )(q, xt, sup, w0, b0r, w1, b1r)


def kernel(support_xyz, support_features, W0_0, b0_0, W0_1, b0_1, W1_0, b1_0,
           W1_1, b1_1, W2_0, b2_0, W2_1, b2_1, W3_0, b3_0, W3_1, b3_1):
    params = [
        (W0_0, b0_0, W0_1, b0_1),
        (W1_0, b1_0, W1_1, b1_1),
        (W2_0, b2_0, W2_1, b2_1),
        (W3_0, b3_0, W3_1, b3_1),
    ]
    xyz = support_xyz
    feat = jnp.transpose(support_features, (0, 2, 1))
    for i in range(4):
        n = xyz.shape[1]
        m = n // _STRIDE
        skey = jax.random.fold_in(jax.random.key(42), i)
        samp = jax.random.permutation(skey, n)[:m]
        q = jnp.take(xyz, samp, axis=1)
        w0, b0, w1, b1 = params[i]
        feat = _run_stage(q, xyz, feat, w0, b0, w1, b1, _RADII[i])
        xyz = q
    return jnp.transpose(feat, (0, 2, 1))


# speedup vs baseline: 12.1570x; 1.1381x over previous
import functools

import jax
import jax.numpy as jnp
from jax.experimental import pallas as pl
from jax.experimental.pallas import tpu as pltpu

_RADII = (0.1, 0.2, 0.4, 0.8)
_NSAMPLE = 32
_STRIDE = 4


def _stage_body(q_ref, xt_ref, s_ref, w0_ref, b0_ref, w1_ref, b1_ref,
                o_ref, a_ref, *, r2, n, nc, mblk, d):
    q = q_ref[0]
    sup = s_ref[0]
    nchunks = n // nc

    i0 = jax.lax.broadcasted_iota(jnp.int32, (nc, nc), 0)
    i1 = jax.lax.broadcasted_iota(jnp.int32, (nc, nc), 1)
    lt = jnp.where(i0 <= i1, 1.0, 0.0).astype(jnp.float32)

    qx = q[:, 0:1]
    qy = q[:, 1:2]
    qz = q[:, 2:3]

    def chunk_step(c, cnt):
        sl = pl.ds(c * nc, nc)
        xr = xt_ref[0, 0:1, sl]
        yr = xt_ref[0, 1:2, sl]
        zr = xt_ref[0, 2:3, sl]
        d2 = (qx - xr) ** 2 + (qy - yr) ** 2 + (qz - zr) ** 2
        mask = d2 <= r2
        maskf = jnp.where(mask, 1.0, 0.0).astype(jnp.float32)
        pos = cnt + jax.lax.dot(maskf, lt,
                                preferred_element_type=jnp.float32)
        a_chunk = jnp.where(mask & (pos <= 32.0), pos - 1.0, -1.0)
        a_ref[:, sl] = a_chunk
        return cnt + jnp.sum(maskf, axis=1, keepdims=True)

    cnt0 = jnp.zeros((mblk, 1), jnp.float32)
    if nchunks == 1:
        cnt = chunk_step(0, cnt0)
    else:
        cnt = jax.lax.fori_loop(0, nchunks, chunk_step, cnt0)

    amap = a_ref[:, :]
    vc = jnp.minimum(cnt, 32.0)
    w0 = w0_ref[:, :]
    b0 = b0_ref[:, :]
    w1 = w1_ref[:, :]
    b1 = b1_ref[:, :]

    def slot_val(s_f, row):
        dp = row[:, 0:3] - q
        xin = jnp.concatenate([dp, row[:, 3:]], axis=1)
        h = jnp.maximum(jnp.dot(xin, w0) + b0, 0.0)
        h = jnp.maximum(jnp.dot(h, w1) + b1, 0.0)
        return jnp.where(s_f < vc, h, 0.0)

    def gather_row(s_f):
        oh = jnp.where(amap == s_f, 1.0, 0.0).astype(jnp.bfloat16)
        p = jax.lax.dot(oh, sup, preferred_element_type=jnp.float32)
        return (p[:, :d] + p[:, d:2 * d]) + p[:, 2 * d:]

    row0 = gather_row(jnp.float32(0.0))
    srow0 = sup[0:1, :].astype(jnp.float32)
    srow0 = (srow0[:, :d] + srow0[:, d:2 * d]) + srow0[:, 2 * d:]
    row0 = jnp.where(cnt < 0.5, srow0, row0)
    acc0 = slot_val(jnp.float32(0.0), row0)

    def s_step(s, acc):
        s_f = s.astype(jnp.float32)
        return jnp.maximum(acc, slot_val(s_f, gather_row(s_f)))

    acc = jax.lax.fori_loop(1, 32, s_step, acc0)
    o_ref[0] = acc


def _run_stage(q, xyz, feat, w0, b0, w1, b1, radius):
    b, m, _ = q.shape
    n = xyz.shape[1]
    c1 = w0.shape[1]
    c2 = w1.shape[1]
    d = 3 + feat.shape[2]

    supf = jnp.concatenate([xyz, feat], axis=2)
    sup_hi = supf.astype(jnp.bfloat16)
    res1 = supf - sup_hi.astype(jnp.float32)
    sup_mid = res1.astype(jnp.bfloat16)
    sup_lo = (res1 - sup_mid.astype(jnp.float32)).astype(jnp.bfloat16)
    sup = jnp.concatenate([sup_hi, sup_mid, sup_lo], axis=2)
    xt = jnp.transpose(xyz, (0, 2, 1))
    b0r = b0.reshape(1, c1)
    b1r = b1.reshape(1, c2)

    mblk = min(m, 512)
    nc = min(n, 512)

    body = functools.partial(_stage_body, r2=radius * radius,
                             n=n, nc=nc, mblk=mblk, d=d)
    return pl.pallas_call(
        body,
        grid=(b, m // mblk),
        in_specs=[
            pl.BlockSpec((1, mblk, 3), lambda i, j: (i, j, 0)),
            pl.BlockSpec((1, 3, n), lambda i, j: (i, 0, 0)),
            pl.BlockSpec((1, n, 3 * d), lambda i, j: (i, 0, 0)),
            pl.BlockSpec((d, c1), lambda i, j: (0, 0)),
            pl.BlockSpec((1, c1), lambda i, j: (0, 0)),
            pl.BlockSpec((c1, c2), lambda i, j: (0, 0)),
            pl.BlockSpec((1, c2), lambda i, j: (0, 0)),
        ],
        out_specs=pl.BlockSpec((1, mblk, c2), lambda i, j: (i, j, 0)),
        out_shape=jax.ShapeDtypeStruct((b, m, c2), jnp.float32),
        scratch_shapes=[pltpu.VMEM((mblk, n), jnp.float32)],
    )(q, xt, sup, w0, b0r, w1, b1r)


def kernel(support_xyz, support_features, W0_0, b0_0, W0_1, b0_1, W1_0, b1_0,
           W1_1, b1_1, W2_0, b2_0, W2_1, b2_1, W3_0, b3_0, W3_1, b3_1):
    params = [
        (W0_0, b0_0, W0_1, b0_1),
        (W1_0, b1_0, W1_1, b1_1),
        (W2_0, b2_0, W2_1, b2_1),
        (W3_0, b3_0, W3_1, b3_1),
    ]
    xyz = support_xyz
    feat = jnp.transpose(support_features, (0, 2, 1))
    for i in range(4):
        n = xyz.shape[1]
        m = n // _STRIDE
        skey = jax.random.fold_in(jax.random.key(42), i)
        samp = jax.random.permutation(skey, n)[:m]
        q = jnp.take(xyz, samp, axis=1)
        w0, b0, w1, b1 = params[i]
        feat = _run_stage(q, xyz, feat, w0, b0, w1, b1, _RADII[i])
        xyz = q
    return jnp.transpose(feat, (0, 2, 1))
